# baseline (speedup 1.0000x reference)
import jax
import jax.numpy as jnp
from jax.experimental import pallas as pl
from jax.experimental.pallas import tpu as pltpu

_BLK = 16384
_OUT_W = 256
_ROWS = _BLK // _OUT_W
_PAD = 8


def _nn_kernel(ea_ref, x_ref, o_ref, xa_ref):
    @pl.when(pl.program_id(0) == 0)
    def _init():
        r = jax.lax.broadcasted_iota(jnp.int32, (_PAD, _BLK), 0)
        xa_ref[64 : 64 + _PAD, :] = (r < 3).astype(jnp.float32)

    xa_ref[0:64, :] = x_ref[...]
    m = jax.lax.dot_general(
        ea_ref[...], xa_ref[...], (((1,), (0,)), ((), ())),
        preferred_element_type=jnp.float32)
    idx = jnp.argmin(m, axis=0).astype(jnp.int32)
    o_ref[...] = idx.reshape(_ROWS, _OUT_W)


def kernel(embedded_data, embedding_matrix):
    b, d, D, H, W = embedded_data.shape
    n = D * H * W
    k = embedding_matrix.shape[0]
    x = embedded_data.reshape(d, n)
    hc2 = 0.5 * jnp.sum(embedding_matrix * embedding_matrix, axis=1,
                        keepdims=True)
    h1 = hc2.astype(jnp.bfloat16).astype(jnp.float32)
    h2 = (hc2 - h1).astype(jnp.bfloat16).astype(jnp.float32)
    h3 = (hc2 - h1 - h2).astype(jnp.bfloat16).astype(jnp.float32)
    ea = jnp.concatenate(
        [-embedding_matrix, h1, h2, h3,
         jnp.zeros((k, _PAD - 3), jnp.float32)],
        axis=1)
    out = pl.pallas_call(
        _nn_kernel,
        grid=(n // _BLK,),
        in_specs=[
            pl.BlockSpec((k, d + _PAD), lambda i: (0, 0)),
            pl.BlockSpec((d, _BLK), lambda i: (0, i)),
        ],
        out_specs=pl.BlockSpec((_ROWS, _OUT_W), lambda i: (i, 0)),
        out_shape=jax.ShapeDtypeStruct((n // _OUT_W, _OUT_W), jnp.int32),
        scratch_shapes=[pltpu.VMEM((d + _PAD, _BLK), jnp.float32)],
    )(ea, x)
    return out.reshape(D, H, W)


# speedup vs baseline: 1.0035x; 1.0035x over previous
import jax
import jax.numpy as jnp
from jax.experimental import pallas as pl
from jax.experimental.pallas import tpu as pltpu

_BLK = 16384
_OUT_W = 256
_ROWS = _BLK // _OUT_W
_PAD = 8
_CHUNKS = 4
_CW = _BLK // _CHUNKS


def _nn_kernel(ea_ref, x_ref, o_ref, xa_ref):
    @pl.when(pl.program_id(0) == 0)
    def _init():
        r = jax.lax.broadcasted_iota(jnp.int32, (_PAD, _BLK), 0)
        xa_ref[64 : 64 + _PAD, :] = (r < 3).astype(jnp.float32)

    xa_ref[0:64, :] = x_ref[...]
    ea = ea_ref[...]
    parts = []
    for c in range(_CHUNKS):
        mc = jax.lax.dot_general(
            ea, xa_ref[:, c * _CW : (c + 1) * _CW],
            (((1,), (0,)), ((), ())),
            preferred_element_type=jnp.float32)
        parts.append(jnp.argmin(mc, axis=0).astype(jnp.int32))
    idx = jnp.concatenate(parts)
    o_ref[...] = idx.reshape(_ROWS, _OUT_W)


def kernel(embedded_data, embedding_matrix):
    b, d, D, H, W = embedded_data.shape
    n = D * H * W
    k = embedding_matrix.shape[0]
    x = embedded_data.reshape(d, n)
    hc2 = 0.5 * jnp.sum(embedding_matrix * embedding_matrix, axis=1,
                        keepdims=True)
    h1 = hc2.astype(jnp.bfloat16).astype(jnp.float32)
    h2 = (hc2 - h1).astype(jnp.bfloat16).astype(jnp.float32)
    h3 = (hc2 - h1 - h2).astype(jnp.bfloat16).astype(jnp.float32)
    ea = jnp.concatenate(
        [-embedding_matrix, h1, h2, h3,
         jnp.zeros((k, _PAD - 3), jnp.float32)],
        axis=1)
    out = pl.pallas_call(
        _nn_kernel,
        grid=(n // _BLK,),
        in_specs=[
            pl.BlockSpec((k, d + _PAD), lambda i: (0, 0)),
            pl.BlockSpec((d, _BLK), lambda i: (0, i)),
        ],
        out_specs=pl.BlockSpec((_ROWS, _OUT_W), lambda i: (i, 0)),
        out_shape=jax.ShapeDtypeStruct((n // _OUT_W, _OUT_W), jnp.int32),
        scratch_shapes=[pltpu.VMEM((d + _PAD, _BLK), jnp.float32)],
    )(ea, x)
    return out.reshape(D, H, W)
